# Initial kernel scaffold; baseline (speedup 1.0000x reference)
import jax
import jax.numpy as jnp
from jax.experimental import pallas as pl


def kernel(inp, table):
    raise NotImplementedError("write your pallas kernel here")


# speedup vs baseline: 1.0150x; 1.0150x over previous
import functools

import jax
import jax.numpy as jnp
from jax import lax
from jax.experimental import pallas as pl
from jax.experimental.pallas import tpu as pltpu
from jax.experimental.pallas import tpu_sc as plsc

B = 16384
L = 50
EMB = 32
N_TOTAL = B * L

NC = 2
NS = 16
NW = NC * NS
N_PER = N_TOTAL // NW

C = 1024
N_CHUNK = N_PER // C
R = 8

_mesh = plsc.VectorSubcoreMesh(core_axis_name="c", subcore_axis_name="s")


@functools.partial(
    pl.kernel,
    mesh=_mesh,
    compiler_params=pltpu.CompilerParams(use_tc_tiling_on_sc=False),
    out_type=jax.ShapeDtypeStruct((N_TOTAL, EMB), jnp.float32),
    scratch_types=[
        pltpu.VMEM((C,), jnp.int32),
        pltpu.VMEM((C, EMB), jnp.float32),
        pltpu.SemaphoreType.DMA,
    ],
)
def _embed_relu(idx_hbm, table_hbm, out_hbm, idx_v, rows_v, sem):
    wid = lax.axis_index("s") * NC + lax.axis_index("c")
    base0 = wid * N_PER

    def chunk_body(c, carry):
        base = base0 + c * C
        pltpu.sync_copy(idx_hbm.at[pl.ds(base, C)], idx_v)
        pltpu.async_copy(table_hbm.at[idx_v], rows_v, sem).wait()

        def relu_body(i, acc):
            for r in range(R):
                row = i * R + r
                for d in range(EMB // 16):
                    sl = pl.ds(d * 16, 16)
                    rows_v[row, sl] = jnp.maximum(rows_v[row, sl], 0.0)
            return acc

        lax.fori_loop(0, C // R, relu_body, 0)
        pltpu.sync_copy(rows_v, out_hbm.at[pl.ds(base, C)])
        return carry

    lax.fori_loop(0, N_CHUNK, chunk_body, 0)


def kernel(inp, table):
    idx = inp.reshape(N_TOTAL).astype(jnp.int32)
    out = _embed_relu(idx, table)
    return out.reshape(B, L, EMB)


# speedup vs baseline: 1.2098x; 1.1920x over previous
import functools

import jax
import jax.numpy as jnp
from jax import lax
from jax.experimental import pallas as pl
from jax.experimental.pallas import tpu as pltpu
from jax.experimental.pallas import tpu_sc as plsc

B = 16384
L = 50
EMB = 32

NC = 2
NS = 16
NW = NC * NS

BB = 128
N_UNITS = L * (B // BB)
U_PER_W = N_UNITS // NW

_mesh = plsc.VectorSubcoreMesh(core_axis_name="c", subcore_axis_name="s")


@functools.partial(
    pl.kernel,
    mesh=_mesh,
    compiler_params=pltpu.CompilerParams(
        use_tc_tiling_on_sc=False, needs_layout_passes=False
    ),
    out_type=jax.ShapeDtypeStruct((L, EMB // 8, B // BB, 8 * BB), jnp.float32),
    scratch_types=[
        pltpu.VMEM((BB,), jnp.int32),
        pltpu.VMEM((BB, EMB), jnp.float32),
        pltpu.VMEM((EMB // 8, 8 * BB), jnp.float32),
        pltpu.SemaphoreType.DMA,
    ],
)
def _embed_relu(idx_hbm, table_hbm, out_hbm, idx_v, rows_v, tile_v, sem):
    wid = lax.axis_index("s") * NC + lax.axis_index("c")
    lane = lax.iota(jnp.int32, 16)

    def unit_body(u, carry):
        unit = wid * U_PER_W + u
        l = unit // (B // BB)
        tb = unit % (B // BB)
        pltpu.sync_copy(idx_hbm.at[l, pl.ds(tb * BB, BB)], idx_v)
        pltpu.async_copy(table_hbm.at[idx_v], rows_v, sem).wait()

        def col_body(e, acc):
            te = e // 8
            ee = e % 8
            for g in range(BB // 16):
                v = plsc.load_gather(rows_v, [g * 16 + lane, jnp.full((16,), e, jnp.int32)])
                tile_v[te, pl.ds(ee * BB + g * 16, 16)] = jnp.maximum(v, 0.0)
            return acc

        lax.fori_loop(0, EMB, col_body, 0)
        pltpu.sync_copy(tile_v, out_hbm.at[l, :, tb])
        return carry

    lax.fori_loop(0, U_PER_W, unit_body, 0)


def kernel(inp, table):
    idx_t = inp.T.astype(jnp.int32)
    out5 = _embed_relu(idx_t, table)
    out = out5.reshape(L, EMB // 8, B // BB, 8, BB)
    out = out.transpose(2, 4, 0, 1, 3)
    return out.reshape(B, L, EMB)


# speedup vs baseline: 1.4351x; 1.1862x over previous
import functools

import jax
import jax.numpy as jnp
from jax import lax
from jax.experimental import pallas as pl
from jax.experimental.pallas import tpu as pltpu
from jax.experimental.pallas import tpu_sc as plsc

B = 16384
L = 50
EMB = 32

NC = 2
NS = 16
NW = NC * NS

TBG = 4
C = TBG * 128
N_UNITS = L * (B // 128) // TBG
U_PER_W = N_UNITS // NW

_mesh = plsc.VectorSubcoreMesh(core_axis_name="c", subcore_axis_name="s")


@functools.partial(
    pl.kernel,
    mesh=_mesh,
    compiler_params=pltpu.CompilerParams(
        use_tc_tiling_on_sc=False, needs_layout_passes=False
    ),
    out_type=jax.ShapeDtypeStruct((L, EMB // 8, B // 128, 1024), jnp.float32),
    scratch_types=[
        pltpu.VMEM((C,), jnp.int32),
        pltpu.VMEM((C,), jnp.int32),
        pltpu.VMEM((C, EMB), jnp.float32),
        pltpu.VMEM((C, EMB), jnp.float32),
        pltpu.VMEM((EMB // 8, TBG, 1024), jnp.float32),
        pltpu.VMEM((EMB // 8, TBG, 1024), jnp.float32),
        pltpu.SemaphoreType.DMA,
        pltpu.SemaphoreType.DMA,
        pltpu.SemaphoreType.DMA,
        pltpu.SemaphoreType.DMA,
    ],
)
def _embed_relu(idx_hbm, table_hbm, out_hbm,
                idx0, idx1, rows0, rows1, tile0, tile1,
                gsem0, gsem1, wsem0, wsem1):
    wid = lax.axis_index("s") * NC + lax.axis_index("c")
    lane = lax.iota(jnp.int32, 16)
    idx_b = (idx0, idx1)
    rows_b = (rows0, rows1)
    tile_b = (tile0, tile1)
    gsem_b = (gsem0, gsem1)
    wsem_b = (wsem0, wsem1)

    def unit_coords(u):
        unit = wid * U_PER_W + u
        l = unit // (B // 128 // TBG)
        tb0 = (unit % (B // 128 // TBG)) * TBG
        return l, tb0

    def start_gather(u, b):
        l, tb0 = unit_coords(u)
        pltpu.sync_copy(idx_hbm.at[l, pl.ds(tb0 * 128, C)], idx_b[b])
        pltpu.async_copy(table_hbm.at[idx_b[b]], rows_b[b], gsem_b[b])

    start_gather(0, 0)

    def unit_body(i, carry):
        for b in range(2):
            u = i * 2 + b
            nb = 1 - b

            @pl.when(u + 1 < U_PER_W)
            def _():
                @pl.when(u >= 1)
                def _():
                    pltpu.make_async_copy(tile_b[nb], out_hbm.at[0, :, pl.ds(0, TBG)],
                                          wsem_b[nb]).wait()

                start_gather(u + 1, nb)

            pltpu.make_async_copy(table_hbm.at[idx_b[b]], rows_b[b], gsem_b[b]).wait()

            def col_body(e, acc):
                te = e // 8
                ee = e % 8
                for j in range(TBG):
                    for g in range(8):
                        v = plsc.load_gather(
                            rows_b[b],
                            [j * 128 + g * 16 + lane, jnp.full((16,), e, jnp.int32)],
                        )
                        tile_b[b][te, j, pl.ds(ee * 128 + g * 16, 16)] = (
                            jnp.maximum(v, 0.0)
                        )
                return acc

            lax.fori_loop(0, EMB, col_body, 0)

            l, tb0 = unit_coords(u)
            pltpu.async_copy(tile_b[b], out_hbm.at[l, :, pl.ds(tb0, TBG)], wsem_b[b])
        return carry

    lax.fori_loop(0, U_PER_W // 2, unit_body, 0)
    pltpu.make_async_copy(tile0, out_hbm.at[0, :, pl.ds(0, TBG)], wsem0).wait()
    pltpu.make_async_copy(tile1, out_hbm.at[0, :, pl.ds(0, TBG)], wsem1).wait()


def kernel(inp, table):
    idx_t = inp.T.astype(jnp.int32)
    out5 = _embed_relu(idx_t, table)
    out = out5.reshape(L, EMB // 8, B // 128, 8, 128)
    out = out.transpose(2, 4, 0, 1, 3)
    return out.reshape(B, L, EMB)


# speedup vs baseline: 2.6274x; 1.8308x over previous
import functools

import jax
import jax.numpy as jnp
from jax import lax
from jax.experimental import pallas as pl
from jax.experimental.pallas import tpu as pltpu
from jax.experimental.pallas import tpu_sc as plsc

B = 16384
L = 50
EMB = 32
BP = 129

NC = 2
NS = 16
NW = NC * NS

TBG = 4
C = TBG * 128
N_UNITS = L * (B // 128) // TBG
U_PER_W = N_UNITS // NW
I_PER_W = U_PER_W * C

_mesh = plsc.VectorSubcoreMesh(core_axis_name="c", subcore_axis_name="s")


@functools.partial(
    pl.kernel,
    mesh=_mesh,
    compiler_params=pltpu.CompilerParams(
        use_tc_tiling_on_sc=False, needs_layout_passes=False
    ),
    out_type=jax.ShapeDtypeStruct((L, EMB // 8, B // 128, 8, 128), jnp.float32),
    scratch_types=[
        pltpu.VMEM((I_PER_W,), jnp.int32),
        pltpu.VMEM((C, EMB), jnp.float32),
        pltpu.VMEM((C, EMB), jnp.float32),
        pltpu.VMEM((EMB // 8, TBG, 8, BP), jnp.float32),
        pltpu.VMEM((EMB // 8, TBG, 8, BP), jnp.float32),
        pltpu.SemaphoreType.DMA,
        pltpu.SemaphoreType.DMA,
        pltpu.SemaphoreType.DMA,
        pltpu.SemaphoreType.DMA,
    ],
)
def _embed_relu(idx_hbm, table_hbm, out_hbm,
                idx_all, rows0, rows1, tile0, tile1,
                gsem0, gsem1, wsem0, wsem1):
    wid = lax.axis_index("s") * NC + lax.axis_index("c")
    lane = lax.iota(jnp.int32, 16)
    te_lo = jax.lax.shift_right_logical(lane, 3)
    te_hi = te_lo + 2
    ee_v = lane & 7
    rows_b = (rows0, rows1)
    tile_b = (tile0, tile1)
    gsem_b = (gsem0, gsem1)
    wsem_b = (wsem0, wsem1)

    pltpu.sync_copy(idx_hbm.at[pl.ds(wid * I_PER_W, I_PER_W)], idx_all)

    def start_gather(u, b):
        pltpu.async_copy(
            table_hbm.at[idx_all.at[pl.ds(u * C, C)]], rows_b[b], gsem_b[b]
        )

    def tile_src(b):
        return tile_b[b].at[:, :, :, pl.ds(0, 128)]

    start_gather(0, 0)

    def unit_body(i, carry):
        for b in range(2):
            u = i * 2 + b
            nb = 1 - b

            @pl.when(u + 1 < U_PER_W)
            def _():
                @pl.when(u >= 1)
                def _():
                    pltpu.make_async_copy(
                        tile_src(nb), out_hbm.at[0, :, pl.ds(0, TBG)], wsem_b[nb]
                    ).wait()

                start_gather(u + 1, nb)

            pltpu.make_async_copy(
                table_hbm.at[idx_all.at[pl.ds(0, C)]], rows_b[b], gsem_b[b]
            ).wait()

            def row_body(rq, acc):
                for s in range(4):
                    r = rq * 4 + s
                    j = jax.lax.shift_right_logical(r, 7)
                    bb = r & 127
                    j_v = jnp.full((16,), 0, jnp.int32) + j
                    bb_v = jnp.full((16,), 0, jnp.int32) + bb
                    v0 = rows_b[b][r, pl.ds(0, 16)]
                    v1 = rows_b[b][r, pl.ds(16, 16)]
                    plsc.store_scatter(
                        tile_b[b], [te_lo, j_v, ee_v, bb_v], jnp.maximum(v0, 0.0)
                    )
                    plsc.store_scatter(
                        tile_b[b], [te_hi, j_v, ee_v, bb_v], jnp.maximum(v1, 0.0)
                    )
                return acc

            lax.fori_loop(0, C // 4, row_body, 0)

            unit = wid * U_PER_W + u
            l = unit // (B // 128 // TBG)
            tb0 = (unit % (B // 128 // TBG)) * TBG
            pltpu.async_copy(
                tile_src(b), out_hbm.at[l, :, pl.ds(tb0, TBG)], wsem_b[b]
            )
        return carry

    lax.fori_loop(0, U_PER_W // 2, unit_body, 0)
    pltpu.make_async_copy(tile_src(0), out_hbm.at[0, :, pl.ds(0, TBG)], wsem0).wait()
    pltpu.make_async_copy(tile_src(1), out_hbm.at[0, :, pl.ds(0, TBG)], wsem1).wait()


def kernel(inp, table):
    idx = inp.T.reshape(L * B).astype(jnp.int32)
    out5 = _embed_relu(idx, table)
    out = out5.transpose(2, 4, 0, 1, 3)
    return out.reshape(B, L, EMB)
